# baseline (speedup 1.0000x reference)
import functools

import jax
import jax.numpy as jnp
from jax.experimental import pallas as pl


def _lrelu(v):
    return jnp.where(v > 0, v, 0.1 * v)


def _dot(a, b):
    return jax.lax.dot_general(
        a, b, (((1,), (0,)), ((), ())), preferred_element_type=jnp.float32
    )


def _taps_kernel(x_ref, *rest, T, offs, relu, head):
    if head:
        *halo_refs, w_ref, wh_ref, o_ref = rest
    else:
        *halo_refs, w_ref, o_ref = rest
    xs = x_ref[0]
    if halo_refs:
        xs = jnp.concatenate([xs] + [h[0] for h in halo_refs], axis=0)
    acc = None
    for k, off in enumerate(offs):
        t = _dot(xs[off : off + T, :], w_ref[k])
        acc = t if acc is None else acc + t
    if relu:
        acc = _lrelu(acc)
    if head:
        acc = _dot(acc, wh_ref[...])
    o_ref[0] = acc


def _run_taps(xf, ws, offs, n_valid, n_halo, R_rows, Wq, relu, head, out_ch):
    B, Npad, C = xf.shape
    if R_rows is None:
        T = n_valid
        grid = (B,)
        in_specs = [pl.BlockSpec((1, Npad, C), lambda b: (b, 0, 0))]
        out_specs = pl.BlockSpec((1, T, out_ch), lambda b: (b, 0, 0))
        halo = 0
    else:
        T = R_rows * Wq
        nt = n_valid // T
        grid = (B, nt)
        in_specs = [pl.BlockSpec((1, T, C), lambda b, i: (b, i, 0))]
        for j in range(n_halo):
            in_specs.append(
                pl.BlockSpec(
                    (1, Wq, C),
                    functools.partial(
                        lambda b, i, jj: (b, (i + 1) * R_rows + jj, 0), jj=j
                    ),
                )
            )
        out_specs = pl.BlockSpec((1, T, out_ch), lambda b, i: (b, i, 0))
        halo = n_halo
    nmap = (lambda b: 0) if R_rows is None else (lambda b, i: 0)
    for w in ws:
        wshape = w.shape
        in_specs.append(
            pl.BlockSpec(wshape, functools.partial(
                lambda *a, n: (0,) * n, n=len(wshape)))
        )
    kfn = functools.partial(_taps_kernel, T=T, offs=offs, relu=relu, head=head)
    return pl.pallas_call(
        kfn,
        grid=grid,
        in_specs=in_specs,
        out_specs=out_specs,
        out_shape=jax.ShapeDtypeStruct((B, n_valid, out_ch), jnp.float32),
    )(xf, *([xf] * halo), *ws)


def _w_s2d(w):
    O, C, _, _ = w.shape
    wp = jnp.pad(w, ((0, 0), (0, 0), (0, 1), (0, 1)))
    wp = wp.reshape(O, C, 2, 2, 2, 2)
    return wp.transpose(2, 4, 3, 5, 1, 0).reshape(4, 4 * C, O)


def _conv3x3_s2(x, w, R_rows=None):
    B, H, W, C = x.shape
    O = w.shape[0]
    H2, W2 = H // 2, W // 2
    Wq = -(-(W2 + 1) // 8) * 8
    t = x.reshape(B, H2, 2, W2, 2, C).transpose(0, 1, 3, 2, 4, 5)
    t = t.reshape(B, H2, W2, 4 * C)
    t = jnp.pad(t, ((0, 0), (0, 2), (0, Wq - W2), (0, 0)))
    xf = t.reshape(B, (H2 + 2) * Wq, 4 * C)
    offs = [a * Wq + b for a in range(2) for b in range(2)]
    out = _run_taps(xf, [_w_s2d(w)], offs, H2 * Wq, 2, R_rows, Wq,
                    relu=True, head=False, out_ch=O)
    return out.reshape(B, H2, Wq, O)[:, :, :W2, :]


def _conv3x3_s1_head(x, wf, wh, R_rows=None):
    B, H, W, C = x.shape
    Wp = -(-(W + 2) // 8) * 8
    xp = jnp.pad(x, ((0, 0), (1, 2), (1, Wp - W - 1), (0, 0)))
    xf = xp.reshape(B, (H + 3) * Wp, C)
    wft = jnp.transpose(wf, (2, 3, 1, 0)).reshape(9, C, wf.shape[0])
    wht = jnp.transpose(wh[:, :, 0, 0], (1, 0))
    offs = [dy * Wp + dx for dy in range(3) for dx in range(3)]
    out = _run_taps(xf, [wft, wht], offs, H * Wp, 3, R_rows, Wp,
                    relu=True, head=True, out_ch=wh.shape[0])
    return out.reshape(B, H, Wp, -1)[:, :, :W, :]


def _mm_kernel(x_ref, w_ref, o_ref):
    o_ref[0] = _dot(x_ref[0], w_ref[...])


def _mm_add_kernel(x_ref, w_ref, u_ref, o_ref):
    o_ref[0] = _dot(x_ref[0], w_ref[...]) + u_ref[0]


def _conv1x1(x, w, u=None, nt=1):
    B, H, W, Cin = x.shape
    wt = jnp.transpose(w[:, :, 0, 0], (1, 0))
    O = wt.shape[1]
    N = H * W
    Nt = N // nt
    xf = x.reshape(B, N, Cin)
    in_specs = [
        pl.BlockSpec((1, Nt, Cin), lambda b, i: (b, i, 0)),
        pl.BlockSpec((Cin, O), lambda b, i: (0, 0)),
    ]
    args = [xf, wt]
    if u is None:
        kfn = _mm_kernel
    else:
        kfn = _mm_add_kernel
        in_specs.append(pl.BlockSpec((1, Nt, O), lambda b, i: (b, i, 0)))
        args.append(u.reshape(B, N, O))
    out = pl.pallas_call(
        kfn,
        grid=(B, nt),
        in_specs=in_specs,
        out_specs=pl.BlockSpec((1, Nt, O), lambda b, i: (b, i, 0)),
        out_shape=jax.ShapeDtypeStruct((B, N, O), jnp.float32),
    )(*args)
    return out.reshape(B, H, W, O)


NUM_CLASSES = 80
NUM_ANCHORS = 3


def _up2(u):
    u = jnp.repeat(u, 2, axis=1)
    return jnp.repeat(u, 2, axis=2)


def _head_out(o_nhwc):
    B, G, _, C = o_nhwc.shape
    o = o_nhwc.reshape(B, G, G, NUM_ANCHORS, 5 + NUM_CLASSES)
    return jnp.transpose(o, (0, 3, 1, 2, 4))


def kernel(x, W1, W2, W3, W4, W5, L3, L4, L5, F3, F4, F5, H3, H4, H5):
    xh = jnp.transpose(x, (0, 2, 3, 1))
    c1 = _conv3x3_s2(xh, W1, R_rows=8)
    c2 = _conv3x3_s2(c1, W2, R_rows=8)
    c3 = _conv3x3_s2(c2, W3, R_rows=13)
    c4 = _conv3x3_s2(c3, W4)
    c5 = _conv3x3_s2(c4, W5)
    p5 = _conv1x1(c5, L5)
    p4 = _conv1x1(c4, L4, u=_up2(p5))
    p3 = _conv1x1(c3, L3, u=_up2(p4), nt=2)
    o3 = _conv3x3_s1_head(p3, F3, H3, R_rows=13)
    o4 = _conv3x3_s1_head(p4, F4, H4)
    o5 = _conv3x3_s1_head(p5, F5, H5)
    return (_head_out(o3), _head_out(o4), _head_out(o5))
